# speedup vs baseline: 8.4606x; 8.4606x over previous
import jax
import jax.numpy as jnp
from jax.experimental import pallas as pl
from jax.experimental.pallas import tpu as pltpu

_HID = 128
_OUT = 256
_FEAT = 16
_N = {'tile': 19, 'vertex': 54, 'edge': 72}
_NPAD = {'tile': 24, 'vertex': 56, 'edge': 72}
_RELS = [
    ('tile', 'tile', 'T2T', True),
    ('tile', 'vertex', 'T2V', False),
    ('tile', 'edge', 'T2E', False),
    ('vertex', 'tile', 'V2T', False),
    ('vertex', 'vertex', 'V2V', True),
    ('vertex', 'edge', 'V2E', False),
    ('edge', 'tile', 'E2T', False),
    ('edge', 'vertex', 'E2V', False),
    ('edge', 'edge', 'E2E', True),
]
_EDGE_COUNTS = {'T2T': 114, 'T2V': 114, 'T2E': 114, 'V2T': 114, 'V2V': 162,
                'V2E': 162, 'E2T': 114, 'E2V': 144, 'E2E': 288}


def _round8(n):
    return (n + 7) // 8 * 8


_EPAD = {}
for _s, _d, _name, _sl in _RELS:
    _e = _EDGE_COUNTS[_name] + (_N[_d] if _sl else 0)
    _EPAD[_name] = _round8(_e)

_HI = jax.lax.Precision.HIGHEST


def _dotT(a, b):
    return jax.lax.dot_general(a, b, (((0,), (0,)), ((), ())), precision=_HI)


def _gat_rel(x_src, x_dst, srcv, dstv, wl, wr, att_row, b_row):
    e_pad = srcv.shape[0]
    hl = jnp.dot(x_src, wl, precision=_HI)
    hr = jnp.dot(x_dst, wr, precision=_HI)
    iota_s = jax.lax.broadcasted_iota(jnp.int32, (e_pad, x_src.shape[0]), 1)
    iota_d = jax.lax.broadcasted_iota(jnp.int32, (e_pad, x_dst.shape[0]), 1)
    gs = jnp.where(srcv == iota_s, 1.0, 0.0)
    gd = jnp.where(dstv == iota_d, 1.0, 0.0)
    el = jnp.dot(gs, hl, precision=_HI)
    ed = jnp.dot(gd, hr, precision=_HI)
    s = el + ed
    e = jnp.where(s > 0, s, 0.2 * s)
    logit = jnp.sum(e * att_row, axis=1, keepdims=True)
    neg = jnp.float32(-jnp.inf)
    masked = jnp.where(gd > 0, logit, neg)
    m_row = jnp.max(masked, axis=0, keepdims=True)
    m_row = jnp.where(m_row > neg, m_row, 0.0)
    mg = jnp.sum(gd * m_row, axis=1, keepdims=True)
    num = jnp.exp(logit - mg)
    den_row = jnp.sum(gd * num, axis=0, keepdims=True)
    deng = jnp.sum(gd * den_row, axis=1, keepdims=True)
    alpha = num / jnp.maximum(deng, 1e-16)
    out = _dotT(gd, alpha * el)
    return out + b_row


def _fwd(xc_t, xc_v, xc_e,
         sT2T, dT2T, sT2V, dT2V, sT2E, dT2E, sV2T, dV2T, sV2V, dV2V,
         sV2E, dV2E, sE2T, dE2T, sE2V, dE2V, sE2E, dE2E,
         c1wl, c1wr, c1att, c1b, c2wl, c2wr, c2att, c2b,
         pw, pb, r1w, r1b, r2w, r2b, out_ref):
    xs = {'tile': xc_t[...], 'vertex': xc_v[...], 'edge': xc_e[...]}
    ei = {'T2T': (sT2T, dT2T), 'T2V': (sT2V, dT2V), 'T2E': (sT2E, dT2E),
          'V2T': (sV2T, dV2T), 'V2V': (sV2V, dV2V), 'V2E': (sV2E, dV2E),
          'E2T': (sE2T, dE2T), 'E2V': (sE2V, dE2V), 'E2E': (sE2E, dE2E)}

    for wl3, wr3, att2, b2 in [(c1wl, c1wr, c1att, c1b),
                               (c2wl, c2wr, c2att, c2b)]:
        acc = {t: jnp.zeros((_NPAD[t], _HID), jnp.float32) for t in _N}
        for i, (s, d, name, _) in enumerate(_RELS):
            acc[d] = acc[d] + _gat_rel(
                xs[s], xs[d], ei[name][0][...], ei[name][1][...],
                wl3[i], wr3[i], att2[i:i + 1, :], b2[i:i + 1, :])
        xs = {t: jnp.where(v > 0, v, jnp.exp(jnp.minimum(v, 0.0)) - 1.0)
              for t, v in acc.items()}

    pooled = []
    for j, t in enumerate(['tile', 'vertex', 'edge']):
        rmask = jax.lax.broadcasted_iota(jnp.int32, (_NPAD[t], 1), 0) < _N[t]
        xm = jnp.where(rmask, xs[t], 0.0)
        mean_x = jnp.sum(xm, axis=0, keepdims=True) / _N[t]
        pooled.append(jnp.dot(mean_x, pw[j], precision=_HI) + pb[j:j + 1, :])
    board = jnp.concatenate(pooled, axis=-1)
    h = jnp.dot(board, r1w[...], precision=_HI) + r1b[...]
    h = jnp.maximum(h, 0.0)
    out_ref[...] = jnp.dot(h, r2w[...], precision=_HI) + r2b[...]


@jax.jit
def kernel(x_tile, x_vertex, x_edge, params,
           ei_T2T, ei_T2V, ei_T2E, ei_V2T, ei_V2V, ei_V2E,
           ei_E2T, ei_E2V, ei_E2E):
    eis = {'T2T': ei_T2T, 'T2V': ei_T2V, 'T2E': ei_T2E, 'V2T': ei_V2T,
           'V2V': ei_V2V, 'V2E': ei_V2E, 'E2T': ei_E2T, 'E2V': ei_E2V,
           'E2E': ei_E2E}
    p = params

    args = []
    for t, x in [('tile', x_tile), ('vertex', x_vertex), ('edge', x_edge)]:
        xc = jnp.concatenate([x, p['emb_' + t]], axis=-1)
        xc = jnp.pad(xc, ((0, _NPAD[t] - _N[t]), (0, 0)))
        args.append(xc)
    for s, d, name, sl in _RELS:
        e = eis[name]
        if sl:
            rng = jnp.arange(_N[d], dtype=e.dtype)
            e = jnp.concatenate([e, jnp.stack([rng, rng])], axis=1)
        pad = _EPAD[name] - e.shape[1]
        e = jnp.pad(e, ((0, 0), (0, pad)), constant_values=-1)
        args.append(e[0].reshape(_EPAD[name], 1))
        args.append(e[1].reshape(_EPAD[name], 1))
    for cv in ['conv1', 'conv2']:
        for f in ['Wl', 'Wr', 'att', 'b']:
            args.append(jnp.stack([p[cv][n][f] for _, _, n, _ in _RELS]))
    args.append(jnp.stack([p['proj'][t]['W'] for t in ['tile', 'vertex', 'edge']]))
    args.append(jnp.stack([p['proj'][t]['b'] for t in ['tile', 'vertex', 'edge']]))
    args.append(p['ro1']['W'])
    args.append(p['ro1']['b'].reshape(1, _OUT))
    args.append(p['ro2']['W'])
    args.append(p['ro2']['b'].reshape(1, _OUT))

    return pl.pallas_call(
        _fwd,
        out_shape=jax.ShapeDtypeStruct((1, _OUT), jnp.float32),
    )(*args)


# speedup vs baseline: 22.7588x; 2.6900x over previous
import jax
import jax.numpy as jnp
from jax.experimental import pallas as pl
from jax.experimental.pallas import tpu as pltpu

_HID = 128
_OUT = 256
_N = {'tile': 19, 'vertex': 54, 'edge': 72}
_RELS = [
    ('tile', 'tile', 'T2T', True),
    ('tile', 'vertex', 'T2V', False),
    ('tile', 'edge', 'T2E', False),
    ('vertex', 'tile', 'V2T', False),
    ('vertex', 'vertex', 'V2V', True),
    ('vertex', 'edge', 'V2E', False),
    ('edge', 'tile', 'E2T', False),
    ('edge', 'vertex', 'E2V', False),
    ('edge', 'edge', 'E2E', True),
]

_HI = jax.lax.Precision.HIGHEST


def _dotT(a, b):
    return jax.lax.dot_general(a, b, (((0,), (0,)), ((), ())), precision=_HI)


def _gat_rel(x_src, x_dst, ei_ref, self_loops, wl, wr, att, b):
    n_src, n_dst = x_src.shape[0], x_dst.shape[0]
    src_row = ei_ref[0:1, :].astype(jnp.float32)
    dst_row = ei_ref[1:2, :].astype(jnp.float32)
    if self_loops:
        loop = jax.lax.broadcasted_iota(jnp.int32, (1, n_dst), 1).astype(jnp.float32)
        src_row = jnp.concatenate([src_row, loop], axis=1)
        dst_row = jnp.concatenate([dst_row, loop], axis=1)
    e_tot = src_row.shape[1]
    ones11 = jnp.ones((1, 1), jnp.float32)
    src_col = _dotT(src_row, ones11)
    dst_col = _dotT(dst_row, ones11)

    hl = jnp.dot(x_src, wl, precision=_HI)
    hr = jnp.dot(x_dst, wr, precision=_HI)
    iota_s = jax.lax.broadcasted_iota(jnp.int32, (e_tot, n_src), 1).astype(jnp.float32)
    iota_d = jax.lax.broadcasted_iota(jnp.int32, (e_tot, n_dst), 1).astype(jnp.float32)
    gs = jnp.where(src_col == iota_s, 1.0, 0.0)
    gd = jnp.where(dst_col == iota_d, 1.0, 0.0)
    el = jnp.dot(gs, hl, precision=_HI)
    ed = jnp.dot(gd, hr, precision=_HI)
    s = el + ed
    e = jnp.where(s > 0, s, 0.2 * s)
    logit = jnp.sum(e * att, axis=1, keepdims=True)
    neg = jnp.float32(-jnp.inf)
    masked = jnp.where(gd > 0, logit, neg)
    m_row = jnp.max(masked, axis=0, keepdims=True)
    m_row = jnp.where(m_row > neg, m_row, 0.0)
    mg = jnp.sum(gd * m_row, axis=1, keepdims=True)
    num = jnp.exp(logit - mg)
    den_row = jnp.sum(gd * num, axis=0, keepdims=True)
    deng = jnp.sum(gd * den_row, axis=1, keepdims=True)
    alpha = num / jnp.maximum(deng, 1e-16)
    return _dotT(gd, alpha * el) + b


def _fwd(*refs):
    it = iter(refs)
    x = {t: next(it)[...] for t in _N}
    emb = {t: next(it)[...] for t in _N}
    ei = {name: next(it) for _, _, name, _ in _RELS}
    conv = []
    for _ in range(2):
        conv.append({name: {f: next(it)[...] for f in ['Wl', 'Wr', 'att', 'b']}
                     for _, _, name, _ in _RELS})
    pw = {t: next(it)[...] for t in _N}
    pb = {t: next(it)[...] for t in _N}
    r1w, r1b, r2w, r2b = (next(it)[...] for _ in range(4))
    out_ref = next(it)

    xs = {t: jnp.concatenate([x[t], emb[t]], axis=-1) for t in _N}
    for cv in conv:
        acc = {t: jnp.zeros((_N[t], _HID), jnp.float32) for t in _N}
        for s, d, name, sl in _RELS:
            p = cv[name]
            acc[d] = acc[d] + _gat_rel(xs[s], xs[d], ei[name], sl,
                                       p['Wl'], p['Wr'], p['att'], p['b'])
        xs = {t: jnp.where(v > 0, v, jnp.exp(jnp.minimum(v, 0.0)) - 1.0)
              for t, v in acc.items()}

    pooled = []
    for t in ['tile', 'vertex', 'edge']:
        mean_x = jnp.sum(xs[t], axis=0, keepdims=True) / _N[t]
        pooled.append(jnp.dot(mean_x, pw[t], precision=_HI) + pb[t])
    board = jnp.concatenate(pooled, axis=-1)
    h = jnp.dot(board, r1w, precision=_HI) + r1b
    h = jnp.maximum(h, 0.0)
    out_ref[...] = jnp.dot(h, r2w, precision=_HI) + r2b


@jax.jit
def kernel(x_tile, x_vertex, x_edge, params,
           ei_T2T, ei_T2V, ei_T2E, ei_V2T, ei_V2V, ei_V2E,
           ei_E2T, ei_E2V, ei_E2E):
    eis = {'T2T': ei_T2T, 'T2V': ei_T2V, 'T2E': ei_T2E, 'V2T': ei_V2T,
           'V2V': ei_V2V, 'V2E': ei_V2E, 'E2T': ei_E2T, 'E2V': ei_E2V,
           'E2E': ei_E2E}
    p = params
    args = [x_tile, x_vertex, x_edge,
            p['emb_tile'], p['emb_vertex'], p['emb_edge']]
    args += [eis[name] for _, _, name, _ in _RELS]
    for cv in ['conv1', 'conv2']:
        for _, _, name, _ in _RELS:
            for f in ['Wl', 'Wr', 'att', 'b']:
                args.append(p[cv][name][f])
    for t in ['tile', 'vertex', 'edge']:
        args.append(p['proj'][t]['W'])
    for t in ['tile', 'vertex', 'edge']:
        args.append(p['proj'][t]['b'])
    args += [p['ro1']['W'], p['ro1']['b'], p['ro2']['W'], p['ro2']['b']]

    return pl.pallas_call(
        _fwd,
        out_shape=jax.ShapeDtypeStruct((1, _OUT), jnp.float32),
    )(*args)


# speedup vs baseline: 32.8834x; 1.4449x over previous
import jax
import jax.numpy as jnp
from jax.experimental import pallas as pl
from jax.experimental.pallas import tpu as pltpu

_HID = 128
_OUT = 256
_N = {'tile': 19, 'vertex': 54, 'edge': 72}
_RELS = [
    ('tile', 'tile', 'T2T', True),
    ('tile', 'vertex', 'T2V', False),
    ('tile', 'edge', 'T2E', False),
    ('vertex', 'tile', 'V2T', False),
    ('vertex', 'vertex', 'V2V', True),
    ('vertex', 'edge', 'V2E', False),
    ('edge', 'tile', 'E2T', False),
    ('edge', 'vertex', 'E2V', False),
    ('edge', 'edge', 'E2E', True),
]

_HI = jax.lax.Precision.DEFAULT


def _dotT(a, b):
    return jax.lax.dot_general(a, b, (((0,), (0,)), ((), ())), precision=_HI)


def _gat_rel(x_src, x_dst, ei_ref, self_loops, wl, wr, att, b):
    n_src, n_dst = x_src.shape[0], x_dst.shape[0]
    src_row = ei_ref[0:1, :].astype(jnp.float32)
    dst_row = ei_ref[1:2, :].astype(jnp.float32)
    if self_loops:
        loop = jax.lax.broadcasted_iota(jnp.int32, (1, n_dst), 1).astype(jnp.float32)
        src_row = jnp.concatenate([src_row, loop], axis=1)
        dst_row = jnp.concatenate([dst_row, loop], axis=1)
    e_tot = src_row.shape[1]
    ones11 = jnp.ones((1, 1), jnp.float32)
    src_col = _dotT(src_row, ones11)
    dst_col = _dotT(dst_row, ones11)

    hl = jnp.dot(x_src, wl, precision=_HI)
    hr = jnp.dot(x_dst, wr, precision=_HI)
    iota_s = jax.lax.broadcasted_iota(jnp.int32, (e_tot, n_src), 1).astype(jnp.float32)
    iota_d = jax.lax.broadcasted_iota(jnp.int32, (e_tot, n_dst), 1).astype(jnp.float32)
    gs = jnp.where(src_col == iota_s, 1.0, 0.0)
    gd = jnp.where(dst_col == iota_d, 1.0, 0.0)
    el = jnp.dot(gs, hl, precision=_HI)
    ed = jnp.dot(gd, hr, precision=_HI)
    s = el + ed
    e = jnp.where(s > 0, s, 0.2 * s)
    logit = jnp.sum(e * att, axis=1, keepdims=True)
    neg = jnp.float32(-jnp.inf)
    masked = jnp.where(gd > 0, logit, neg)
    m_row = jnp.max(masked, axis=0, keepdims=True)
    m_row = jnp.where(m_row > neg, m_row, 0.0)
    mg = jnp.sum(gd * m_row, axis=1, keepdims=True)
    num = jnp.exp(logit - mg)
    den_row = jnp.sum(gd * num, axis=0, keepdims=True)
    deng = jnp.sum(gd * den_row, axis=1, keepdims=True)
    alpha = num / jnp.maximum(deng, 1e-16)
    return _dotT(gd, alpha * el) + b


def _fwd(*refs):
    it = iter(refs)
    x = {t: next(it)[...] for t in _N}
    emb = {t: next(it)[...] for t in _N}
    ei = {name: next(it) for _, _, name, _ in _RELS}
    conv = []
    for _ in range(2):
        conv.append({name: {f: next(it)[...] for f in ['Wl', 'Wr', 'att', 'b']}
                     for _, _, name, _ in _RELS})
    pw = {t: next(it)[...] for t in _N}
    pb = {t: next(it)[...] for t in _N}
    r1w, r1b, r2w, r2b = (next(it)[...] for _ in range(4))
    out_ref = next(it)

    xs = {t: jnp.concatenate([x[t], emb[t]], axis=-1) for t in _N}
    for cv in conv:
        acc = {t: jnp.zeros((_N[t], _HID), jnp.float32) for t in _N}
        for s, d, name, sl in _RELS:
            p = cv[name]
            acc[d] = acc[d] + _gat_rel(xs[s], xs[d], ei[name], sl,
                                       p['Wl'], p['Wr'], p['att'], p['b'])
        xs = {t: jnp.where(v > 0, v, jnp.exp(jnp.minimum(v, 0.0)) - 1.0)
              for t, v in acc.items()}

    pooled = []
    for t in ['tile', 'vertex', 'edge']:
        mean_x = jnp.sum(xs[t], axis=0, keepdims=True) / _N[t]
        pooled.append(jnp.dot(mean_x, pw[t], precision=_HI) + pb[t])
    board = jnp.concatenate(pooled, axis=-1)
    h = jnp.dot(board, r1w, precision=_HI) + r1b
    h = jnp.maximum(h, 0.0)
    out_ref[...] = jnp.dot(h, r2w, precision=_HI) + r2b


@jax.jit
def kernel(x_tile, x_vertex, x_edge, params,
           ei_T2T, ei_T2V, ei_T2E, ei_V2T, ei_V2V, ei_V2E,
           ei_E2T, ei_E2V, ei_E2E):
    eis = {'T2T': ei_T2T, 'T2V': ei_T2V, 'T2E': ei_T2E, 'V2T': ei_V2T,
           'V2V': ei_V2V, 'V2E': ei_V2E, 'E2T': ei_E2T, 'E2V': ei_E2V,
           'E2E': ei_E2E}
    p = params
    args = [x_tile, x_vertex, x_edge,
            p['emb_tile'], p['emb_vertex'], p['emb_edge']]
    args += [eis[name] for _, _, name, _ in _RELS]
    for cv in ['conv1', 'conv2']:
        for _, _, name, _ in _RELS:
            for f in ['Wl', 'Wr', 'att', 'b']:
                args.append(p[cv][name][f])
    for t in ['tile', 'vertex', 'edge']:
        args.append(p['proj'][t]['W'])
    for t in ['tile', 'vertex', 'edge']:
        args.append(p['proj'][t]['b'])
    args += [p['ro1']['W'], p['ro1']['b'], p['ro2']['W'], p['ro2']['b']]

    return pl.pallas_call(
        _fwd,
        out_shape=jax.ShapeDtypeStruct((1, _OUT), jnp.float32),
    )(*args)
